# baseline (speedup 1.0000x reference)
import functools
import math

import jax
import jax.numpy as jnp
import numpy as np
from jax.experimental import pallas as pl
from jax.experimental.pallas import tpu as pltpu

S, D = 2048, 2048
H, KVH, HD = 16, 4, 128
GQ = H // KVH
BLK = 64
GH = 128
I = 5504
EPS = 1e-6
THRESH = 1e-3
THETA = 10000.0
NB = S // BLK

_F32 = jnp.float32
_BF16 = jnp.bfloat16


def _rmsnorm_body(x_ref, g_ref, o_ref):
    x = x_ref[...]
    var = jnp.mean(x * x, axis=-1, keepdims=True)
    o_ref[...] = ((x * jax.lax.rsqrt(var + EPS)) * g_ref[...]).astype(_BF16)


def _rmsnorm(x, g, bm=512):
    m, k = x.shape
    return pl.pallas_call(
        _rmsnorm_body,
        grid=(m // bm,),
        in_specs=[
            pl.BlockSpec((bm, k), lambda mm: (mm, 0)),
            pl.BlockSpec((1, k), lambda mm: (0, 0)),
        ],
        out_specs=pl.BlockSpec((bm, k), lambda mm: (mm, 0)),
        out_shape=jax.ShapeDtypeStruct((m, k), _BF16),
    )(x, g.reshape(1, k))


_QKV_BM = 512
_PB = _QKV_BM // BLK


def _rope_piece(x, c, sn):
    x1 = x[:, :HD // 2]
    x2 = x[:, HD // 2:]
    rot = jnp.concatenate([-x2, x1], axis=-1)
    return x * c + rot * sn


def _qkv_body(x_ref, qw_ref, kw_ref, vw_ref, b_ref, cos_ref, sin_ref,
              g_ref, o_ref, p_ref):
    xr = x_ref[...]
    var = jnp.mean(xr * xr, axis=-1, keepdims=True)
    x = ((xr * jax.lax.rsqrt(var + EPS)) * g_ref[...]).astype(_BF16)
    qw = qw_ref[...].astype(_BF16)
    kw = kw_ref[...].astype(_BF16)
    vw = vw_ref[...].astype(_BF16)
    oq = jnp.dot(x, qw, preferred_element_type=_F32)
    ok = jnp.dot(x, kw, preferred_element_type=_F32)
    ov = jnp.dot(x, vw, preferred_element_type=_F32)
    out = jnp.concatenate([oq, ok, ov], axis=-1) + b_ref[...]
    p_ref[...] = out.reshape(_PB, BLK, (H + 2 * KVH) * HD).sum(axis=1)
    c = cos_ref[...]
    sn = sin_ref[...]
    for hh in range(H + KVH):
        piece = out[:, hh * HD:(hh + 1) * HD]
        o_ref[:, hh * HD:(hh + 1) * HD] = _rope_piece(piece, c, sn).astype(_BF16)
    o_ref[:, (H + KVH) * HD:] = out[:, (H + KVH) * HD:].astype(_BF16)


def _qkv(x, qw, kw, vw, b, cos, sin, g):
    n_all = (H + 2 * KVH) * HD
    return pl.pallas_call(
        _qkv_body,
        grid=(S // _QKV_BM,),
        in_specs=[
            pl.BlockSpec((_QKV_BM, D), lambda mm: (mm, 0)),
            pl.BlockSpec((D, H * HD), lambda mm: (0, 0)),
            pl.BlockSpec((D, KVH * HD), lambda mm: (0, 0)),
            pl.BlockSpec((D, KVH * HD), lambda mm: (0, 0)),
            pl.BlockSpec((1, n_all), lambda mm: (0, 0)),
            pl.BlockSpec((_QKV_BM, HD), lambda mm: (mm, 0)),
            pl.BlockSpec((_QKV_BM, HD), lambda mm: (mm, 0)),
            pl.BlockSpec((1, D), lambda mm: (0, 0)),
        ],
        out_specs=[
            pl.BlockSpec((_QKV_BM, n_all), lambda mm: (mm, 0)),
            pl.BlockSpec((_PB, n_all), lambda mm: (mm, 0)),
        ],
        out_shape=[
            jax.ShapeDtypeStruct((S, n_all), _BF16),
            jax.ShapeDtypeStruct((NB, n_all), _F32),
        ],
    )(x, qw, kw, vw, b.reshape(1, n_all), cos, sin, g.reshape(1, D))


def _matmul_res_body(x_ref, w_ref, r_ref, o_ref):
    w = w_ref[...].astype(_BF16)
    o_ref[...] = r_ref[...] + jnp.dot(
        x_ref[...], w, preferred_element_type=_F32)


def _matmul_res(x, w, r, bm, bn):
    m, k = x.shape
    n = w.shape[1]
    grid = (pl.cdiv(n, bn), pl.cdiv(m, bm))
    return pl.pallas_call(
        _matmul_res_body,
        grid=grid,
        in_specs=[
            pl.BlockSpec((bm, k), lambda nn, mm: (mm, 0)),
            pl.BlockSpec((k, bn), lambda nn, mm: (0, nn)),
            pl.BlockSpec((bm, bn), lambda nn, mm: (mm, nn)),
        ],
        out_specs=pl.BlockSpec((bm, bn), lambda nn, mm: (mm, nn)),
        out_shape=jax.ShapeDtypeStruct((m, n), _F32),
    )(x, w, r)


def _mlp1_body(x_ref, gw_ref, uw_ref, o_ref):
    x = x_ref[...]
    a = jnp.dot(x, gw_ref[...].astype(_BF16), preferred_element_type=_F32)
    u = jnp.dot(x, uw_ref[...].astype(_BF16), preferred_element_type=_F32)
    o_ref[...] = ((a * jax.nn.sigmoid(a)) * u).astype(_BF16)


def _mlp1(x, gw, uw, bm, bn):
    m, k = x.shape
    n = gw.shape[1]
    grid = (pl.cdiv(n, bn), pl.cdiv(m, bm))
    return pl.pallas_call(
        _mlp1_body,
        grid=grid,
        in_specs=[
            pl.BlockSpec((bm, k), lambda nn, mm: (mm, 0)),
            pl.BlockSpec((k, bn), lambda nn, mm: (0, nn)),
            pl.BlockSpec((k, bn), lambda nn, mm: (0, nn)),
        ],
        out_specs=pl.BlockSpec((bm, bn), lambda nn, mm: (mm, nn)),
        out_shape=jax.ShapeDtypeStruct((m, n), _BF16),
    )(x, gw, uw)


BQ = 256
BQB = BQ // BLK
BKV = 512
MQ = S // BQ
_SCALE = 1.0 / math.sqrt(HD)
_NEG = -1e9


def _flash_body(q_ref, k_ref, v_ref, b_ref, e_ref, o_ref):
    mi = pl.program_id(1)
    jlast = mi // 2

    for h in range(GQ):
        q = q_ref[:, h * HD:(h + 1) * HD]
        bb = b_ref[0, h, 0]

        def chunk(jj, carry, causal):
            m_prev, l_prev, acc = carry
            kc = k_ref[pl.ds(jj * BKV, BKV), :]
            s = jax.lax.dot_general(
                q, kc, (((1,), (1,)), ((), ())),
                preferred_element_type=_F32) * _SCALE
            ec = e_ref[:, pl.ds(jj * BKV, BKV)]
            bc = jnp.dot(bb, ec, preferred_element_type=_F32)
            s = (s.reshape(BQB, BLK, BKV) + bc[:, None, :]).reshape(BQ, BKV)
            if causal:
                rows = mi * BQ + jax.lax.broadcasted_iota(
                    jnp.int32, (BQ, BKV), 0)
                cols = jj * BKV + jax.lax.broadcasted_iota(
                    jnp.int32, (BQ, BKV), 1)
                s = jnp.where(cols <= rows, s, _NEG)
            m_new = jnp.maximum(m_prev, jnp.max(s, axis=-1, keepdims=True))
            p = jnp.exp(s - m_new)
            alpha = jnp.exp(m_prev - m_new)
            l_new = l_prev * alpha + jnp.sum(p, axis=-1, keepdims=True)
            vc = v_ref[pl.ds(jj * BKV, BKV), :]
            acc_new = acc * alpha + jnp.dot(
                p.astype(_BF16), vc, preferred_element_type=_F32)
            return m_new, l_new, acc_new

        init = (jnp.full((BQ, 1), -1e30, _F32),
                jnp.zeros((BQ, 1), _F32),
                jnp.zeros((BQ, HD), _F32))
        carry = jax.lax.fori_loop(
            0, jlast, lambda jj, c: chunk(jj, c, causal=False), init)
        _, l_fin, acc_fin = chunk(jlast, carry, causal=True)
        o_ref[:, h * HD:(h + 1) * HD] = (acc_fin / l_fin).astype(_BF16)


def _flash(qkv, bias, expander):
    grid = (KVH, MQ)
    return pl.pallas_call(
        _flash_body,
        grid=grid,
        in_specs=[
            pl.BlockSpec((BQ, GQ * HD), lambda g, m: (m, g)),
            pl.BlockSpec((S, HD), lambda g, m: (0, H + g)),
            pl.BlockSpec((S, HD), lambda g, m: (0, H + KVH + g)),
            pl.BlockSpec((1, GQ, 1, BQB, NB), lambda g, m: (g, 0, m, 0, 0)),
            pl.BlockSpec((NB, S), lambda g, m: (0, 0)),
        ],
        out_specs=pl.BlockSpec((BQ, GQ * HD), lambda g, m: (m, g)),
        out_shape=jax.ShapeDtypeStruct((S, H * HD), _BF16),
    )(qkv, qkv, qkv, bias, expander)


def _gate_bias(pooled):
    q_pool = pooled[:, :H * HD].reshape(NB, H, HD) / BLK
    k_pool = pooled[:, H * HD:(H + KVH) * HD].reshape(NB, KVH, HD) / BLK
    return q_pool, k_pool


def _gate_mask_bias(q_pool, k_pool, gq_w, gk_w):
    qg = jnp.einsum('nhd,dg->nhg', q_pool, gq_w,
                    precision=jax.lax.Precision.HIGHEST)
    kg = jnp.einsum('nhd,dg->nhg', k_pool, gk_w,
                    precision=jax.lax.Precision.HIGHEST)
    kg = jnp.repeat(kg, GQ, axis=1)
    logits = jnp.einsum('qhg,khg->hqk', qg, kg,
                        precision=jax.lax.Precision.HIGHEST) / np.sqrt(GH)
    blk_causal = jnp.tril(jnp.ones((NB, NB), dtype=bool))
    logits = jnp.where(blk_causal[None], logits, _NEG)
    score = jax.nn.softmax(logits, axis=-1)
    diag = jnp.eye(NB, dtype=bool)
    mask = ((score >= THRESH) | diag[None]) & blk_causal[None]
    return jnp.where(mask, 0.0, _NEG).astype(_F32)


def _rope_tables(position_ids):
    inv_freq = 1.0 / (THETA ** (jnp.arange(0, HD, 2, dtype=_F32) / HD))
    freqs = position_ids[0].astype(_F32)[:, None] * inv_freq[None, :]
    emb = jnp.concatenate([freqs, freqs], axis=-1)
    return jnp.cos(emb), jnp.sin(emb)


def _rope(x, cos, sin):
    x1, x2 = jnp.split(x, 2, axis=-1)
    rot = jnp.concatenate([-x2, x1], axis=-1)
    return x * cos[:, None, :] + rot * sin[:, None, :]


def kernel(hidden_states, position_ids, ln1_w, q_w, q_b, k_w, k_b, v_w, v_b,
           o_w, gq_w, gk_w, ln2_w, gate_w, up_w, down_w):
    hs = hidden_states.reshape(S, D)

    cos, sin = _rope_tables(position_ids)
    bqkv = jnp.concatenate([q_b, k_b, v_b], axis=0)
    qkv, pooled = _qkv(hs, q_w, k_w, v_w, bqkv, cos, sin, ln1_w)

    q_pool, k_pool = _gate_bias(pooled)
    bias_blk = _gate_mask_bias(q_pool, k_pool, gq_w, gk_w)
    bias5 = bias_blk.reshape(KVH, GQ, MQ, BQB, NB)
    expander = jnp.equal(jnp.arange(S, dtype=jnp.int32)[None, :] // BLK,
                         jnp.arange(NB, dtype=jnp.int32)[:, None]
                         ).astype(_F32)

    attn2 = _flash(qkv, bias5, expander)

    hidden = _matmul_res(attn2, o_w, hs, bm=512, bn=1024)

    xn2 = _rmsnorm(hidden, ln2_w)
    mlp_mid = _mlp1(xn2, gate_w, up_w, bm=1024, bn=512)
    out = _matmul_res(mlp_mid, down_w, hidden, bm=512, bn=512)
    return out.reshape(1, S, D)


# speedup vs baseline: 1.0393x; 1.0393x over previous
import functools
import math

import jax
import jax.numpy as jnp
import numpy as np
from jax.experimental import pallas as pl
from jax.experimental.pallas import tpu as pltpu

S, D = 2048, 2048
H, KVH, HD = 16, 4, 128
GQ = H // KVH
BLK = 64
GH = 128
I = 5504
EPS = 1e-6
THRESH = 1e-3
THETA = 10000.0
NB = S // BLK

_F32 = jnp.float32
_BF16 = jnp.bfloat16


def _rmsnorm_body(x_ref, g_ref, o_ref):
    x = x_ref[...]
    var = jnp.mean(x * x, axis=-1, keepdims=True)
    o_ref[...] = ((x * jax.lax.rsqrt(var + EPS)) * g_ref[...]).astype(_BF16)


def _rmsnorm(x, g, bm=512):
    m, k = x.shape
    return pl.pallas_call(
        _rmsnorm_body,
        grid=(m // bm,),
        in_specs=[
            pl.BlockSpec((bm, k), lambda mm: (mm, 0)),
            pl.BlockSpec((1, k), lambda mm: (0, 0)),
        ],
        out_specs=pl.BlockSpec((bm, k), lambda mm: (mm, 0)),
        out_shape=jax.ShapeDtypeStruct((m, k), _BF16),
    )(x, g.reshape(1, k))


_QKV_BM = 512
_PB = _QKV_BM // BLK


def _rope_piece(x, c, sn):
    x1 = x[:, :HD // 2]
    x2 = x[:, HD // 2:]
    rot = jnp.concatenate([-x2, x1], axis=-1)
    return x * c + rot * sn


def _qkv_body(x_ref, qw_ref, kw_ref, vw_ref, b_ref, cos_ref, sin_ref,
              g_ref, o_ref, p_ref):
    xr = x_ref[...]
    var = jnp.mean(xr * xr, axis=-1, keepdims=True)
    x = ((xr * jax.lax.rsqrt(var + EPS)) * g_ref[...]).astype(_BF16)
    qw = qw_ref[...].astype(_BF16)
    kw = kw_ref[...].astype(_BF16)
    vw = vw_ref[...].astype(_BF16)
    oq = jnp.dot(x, qw, preferred_element_type=_F32)
    ok = jnp.dot(x, kw, preferred_element_type=_F32)
    ov = jnp.dot(x, vw, preferred_element_type=_F32)
    out = jnp.concatenate([oq, ok, ov], axis=-1) + b_ref[...]
    p_ref[...] = out.reshape(_PB, BLK, (H + 2 * KVH) * HD).sum(axis=1)
    c = cos_ref[...]
    sn = sin_ref[...]
    for hh in range(H + KVH):
        piece = out[:, hh * HD:(hh + 1) * HD]
        o_ref[:, hh * HD:(hh + 1) * HD] = _rope_piece(piece, c, sn).astype(_BF16)
    o_ref[:, (H + KVH) * HD:] = out[:, (H + KVH) * HD:].astype(_BF16)


def _qkv(x, qw, kw, vw, b, cos, sin, g):
    n_all = (H + 2 * KVH) * HD
    return pl.pallas_call(
        _qkv_body,
        grid=(S // _QKV_BM,),
        in_specs=[
            pl.BlockSpec((_QKV_BM, D), lambda mm: (mm, 0)),
            pl.BlockSpec((D, H * HD), lambda mm: (0, 0)),
            pl.BlockSpec((D, KVH * HD), lambda mm: (0, 0)),
            pl.BlockSpec((D, KVH * HD), lambda mm: (0, 0)),
            pl.BlockSpec((1, n_all), lambda mm: (0, 0)),
            pl.BlockSpec((_QKV_BM, HD), lambda mm: (mm, 0)),
            pl.BlockSpec((_QKV_BM, HD), lambda mm: (mm, 0)),
            pl.BlockSpec((1, D), lambda mm: (0, 0)),
        ],
        out_specs=[
            pl.BlockSpec((_QKV_BM, n_all), lambda mm: (mm, 0)),
            pl.BlockSpec((_PB, n_all), lambda mm: (mm, 0)),
        ],
        out_shape=[
            jax.ShapeDtypeStruct((S, n_all), _BF16),
            jax.ShapeDtypeStruct((NB, n_all), _F32),
        ],
    )(x, qw, kw, vw, b.reshape(1, n_all), cos, sin, g.reshape(1, D))


def _matmul_res_body(x_ref, w_ref, r_ref, o_ref):
    w = w_ref[...].astype(_BF16)
    o_ref[...] = r_ref[...] + jnp.dot(
        x_ref[...], w, preferred_element_type=_F32)


def _matmul_res(x, w, r, bm, bn):
    m, k = x.shape
    n = w.shape[1]
    grid = (pl.cdiv(n, bn), pl.cdiv(m, bm))
    return pl.pallas_call(
        _matmul_res_body,
        grid=grid,
        in_specs=[
            pl.BlockSpec((bm, k), lambda nn, mm: (mm, 0)),
            pl.BlockSpec((k, bn), lambda nn, mm: (0, nn)),
            pl.BlockSpec((bm, bn), lambda nn, mm: (mm, nn)),
        ],
        out_specs=pl.BlockSpec((bm, bn), lambda nn, mm: (mm, nn)),
        out_shape=jax.ShapeDtypeStruct((m, n), _F32),
    )(x, w, r)


def _mlp1_body(x_ref, gw_ref, uw_ref, o_ref):
    x = x_ref[...]
    a = jnp.dot(x, gw_ref[...].astype(_BF16), preferred_element_type=_F32)
    u = jnp.dot(x, uw_ref[...].astype(_BF16), preferred_element_type=_F32)
    o_ref[...] = ((a * jax.nn.sigmoid(a)) * u).astype(_BF16)


def _mlp1(x, gw, uw, bm, bn):
    m, k = x.shape
    n = gw.shape[1]
    grid = (pl.cdiv(n, bn), pl.cdiv(m, bm))
    return pl.pallas_call(
        _mlp1_body,
        grid=grid,
        in_specs=[
            pl.BlockSpec((bm, k), lambda nn, mm: (mm, 0)),
            pl.BlockSpec((k, bn), lambda nn, mm: (0, nn)),
            pl.BlockSpec((k, bn), lambda nn, mm: (0, nn)),
        ],
        out_specs=pl.BlockSpec((bm, bn), lambda nn, mm: (mm, nn)),
        out_shape=jax.ShapeDtypeStruct((m, n), _BF16),
    )(x, gw, uw)


BQ = 256
BQB = BQ // BLK
BKV = 512
MQ = S // BQ
_SCALE = 1.0 / math.sqrt(HD)
_NEG = -1e9


def _flash_body(q_ref, k_ref, v_ref, b_ref, o_ref):
    mi = pl.program_id(1)
    jlast = mi // 2

    for h in range(GQ):
        q = q_ref[:, h * HD:(h + 1) * HD]

        def chunk(jj, carry, causal):
            m_prev, l_prev, acc = carry
            kc = k_ref[pl.ds(jj * BKV, BKV), :]
            s = jax.lax.dot_general(
                q, kc, (((1,), (1,)), ((), ())),
                preferred_element_type=_F32) * _SCALE
            bc = b_ref[0, h, 0, :, pl.ds(jj * BKV, BKV)]
            s = (s.reshape(BQB, BLK, BKV) + bc[:, None, :]).reshape(BQ, BKV)
            if causal:
                rows = mi * BQ + jax.lax.broadcasted_iota(
                    jnp.int32, (BQ, BKV), 0)
                cols = jj * BKV + jax.lax.broadcasted_iota(
                    jnp.int32, (BQ, BKV), 1)
                s = jnp.where(cols <= rows, s, _NEG)
            m_new = jnp.maximum(m_prev, jnp.max(s, axis=-1, keepdims=True))
            p = jnp.exp(s - m_new)
            alpha = jnp.exp(m_prev - m_new)
            l_new = l_prev * alpha + jnp.sum(p, axis=-1, keepdims=True)
            vc = v_ref[pl.ds(jj * BKV, BKV), :]
            acc_new = acc * alpha + jnp.dot(
                p.astype(_BF16), vc, preferred_element_type=_F32)
            return m_new, l_new, acc_new

        init = (jnp.full((BQ, 1), -1e30, _F32),
                jnp.zeros((BQ, 1), _F32),
                jnp.zeros((BQ, HD), _F32))
        carry = jax.lax.fori_loop(
            0, jlast, lambda jj, c: chunk(jj, c, causal=False), init)
        _, l_fin, acc_fin = chunk(jlast, carry, causal=True)
        o_ref[:, h * HD:(h + 1) * HD] = (acc_fin / l_fin).astype(_BF16)


def _flash(qkv, bias):
    grid = (KVH, MQ)
    return pl.pallas_call(
        _flash_body,
        grid=grid,
        in_specs=[
            pl.BlockSpec((BQ, GQ * HD), lambda g, m: (m, g)),
            pl.BlockSpec((S, HD), lambda g, m: (0, H + g)),
            pl.BlockSpec((S, HD), lambda g, m: (0, H + KVH + g)),
            pl.BlockSpec((1, GQ, 1, BQB, S), lambda g, m: (g, 0, m, 0, 0)),
        ],
        out_specs=pl.BlockSpec((BQ, GQ * HD), lambda g, m: (m, g)),
        out_shape=jax.ShapeDtypeStruct((S, H * HD), _BF16),
    )(qkv, qkv, qkv, bias)


def _gate_bias(pooled):
    q_pool = pooled[:, :H * HD].reshape(NB, H, HD) / BLK
    k_pool = pooled[:, H * HD:(H + KVH) * HD].reshape(NB, KVH, HD) / BLK
    return q_pool, k_pool


def _gate_mask_bias(q_pool, k_pool, gq_w, gk_w):
    qg = jnp.einsum('nhd,dg->nhg', q_pool, gq_w,
                    precision=jax.lax.Precision.HIGHEST)
    kg = jnp.einsum('nhd,dg->nhg', k_pool, gk_w,
                    precision=jax.lax.Precision.HIGHEST)
    kg = jnp.repeat(kg, GQ, axis=1)
    logits = jnp.einsum('qhg,khg->hqk', qg, kg,
                        precision=jax.lax.Precision.HIGHEST) / np.sqrt(GH)
    blk_causal = jnp.tril(jnp.ones((NB, NB), dtype=bool))
    logits = jnp.where(blk_causal[None], logits, _NEG)
    score = jax.nn.softmax(logits, axis=-1)
    diag = jnp.eye(NB, dtype=bool)
    mask = ((score >= THRESH) | diag[None]) & blk_causal[None]
    bias = jnp.where(mask, 0.0, _NEG).astype(_F32)
    return jnp.repeat(bias, BLK, axis=2)


def _rope_tables(position_ids):
    inv_freq = 1.0 / (THETA ** (jnp.arange(0, HD, 2, dtype=_F32) / HD))
    freqs = position_ids[0].astype(_F32)[:, None] * inv_freq[None, :]
    emb = jnp.concatenate([freqs, freqs], axis=-1)
    return jnp.cos(emb), jnp.sin(emb)


def _rope(x, cos, sin):
    x1, x2 = jnp.split(x, 2, axis=-1)
    rot = jnp.concatenate([-x2, x1], axis=-1)
    return x * cos[:, None, :] + rot * sin[:, None, :]


def kernel(hidden_states, position_ids, ln1_w, q_w, q_b, k_w, k_b, v_w, v_b,
           o_w, gq_w, gk_w, ln2_w, gate_w, up_w, down_w):
    hs = hidden_states.reshape(S, D)

    cos, sin = _rope_tables(position_ids)
    bqkv = jnp.concatenate([q_b, k_b, v_b], axis=0)
    qkv, pooled = _qkv(hs, q_w, k_w, v_w, bqkv, cos, sin, ln1_w)

    q_pool, k_pool = _gate_bias(pooled)
    bias_tok = _gate_mask_bias(q_pool, k_pool, gq_w, gk_w)
    bias5 = bias_tok.reshape(KVH, GQ, MQ, BQB, S)

    attn2 = _flash(qkv, bias5)

    hidden = _matmul_res(attn2, o_w, hs, bm=512, bn=1024)

    xn2 = _rmsnorm(hidden, ln2_w)
    mlp_mid = _mlp1(xn2, gate_w, up_w, bm=1024, bn=512)
    out = _matmul_res(mlp_mid, down_w, hidden, bm=512, bn=512)
    return out.reshape(1, S, D)


# speedup vs baseline: 1.0547x; 1.0148x over previous
import functools
import math

import jax
import jax.numpy as jnp
import numpy as np
from jax.experimental import pallas as pl
from jax.experimental.pallas import tpu as pltpu

S, D = 2048, 2048
H, KVH, HD = 16, 4, 128
GQ = H // KVH
BLK = 64
GH = 128
I = 5504
EPS = 1e-6
THRESH = 1e-3
THETA = 10000.0
NB = S // BLK

_F32 = jnp.float32
_BF16 = jnp.bfloat16


def _gate_body(p_ref, gqw_ref, gkw_ref, e_ref, o_ref):
    inv = 1.0 / BLK
    gkw = gkw_ref[...]
    gqw = gqw_ref[...]
    ex = e_ref[...]
    kgs = []
    for g in range(KVH):
        pk = p_ref[:, (H + g) * HD:(H + g + 1) * HD] * inv
        kgs.append(jnp.dot(pk, gkw, preferred_element_type=_F32))
    ri = jax.lax.broadcasted_iota(jnp.int32, (NB, NB), 0)
    ci = jax.lax.broadcasted_iota(jnp.int32, (NB, NB), 1)
    causal = ci <= ri
    diag = ci == ri
    for h in range(H):
        pq = p_ref[:, h * HD:(h + 1) * HD] * inv
        qg = jnp.dot(pq, gqw, preferred_element_type=_F32)
        lg = jax.lax.dot_general(
            qg, kgs[h // GQ], (((1,), (1,)), ((), ())),
            preferred_element_type=_F32) * (1.0 / math.sqrt(GH))
        lg = jnp.where(causal, lg, _NEG)
        mx = jnp.max(lg, axis=-1, keepdims=True)
        pe = jnp.exp(lg - mx)
        score = pe / jnp.sum(pe, axis=-1, keepdims=True)
        mask = ((score >= THRESH) | diag) & causal
        bias = jnp.where(mask, 0.0, _NEG)
        o_ref[h] = jnp.dot(bias, ex, preferred_element_type=_F32)


def _gate(pooled, gq_w, gk_w, expander):
    n_all = (H + 2 * KVH) * HD
    return pl.pallas_call(
        _gate_body,
        grid=(1,),
        in_specs=[
            pl.BlockSpec((NB, n_all), lambda i: (0, 0)),
            pl.BlockSpec((HD, GH), lambda i: (0, 0)),
            pl.BlockSpec((HD, GH), lambda i: (0, 0)),
            pl.BlockSpec((NB, S), lambda i: (0, 0)),
        ],
        out_specs=pl.BlockSpec((H, NB, S), lambda i: (0, 0, 0)),
        out_shape=jax.ShapeDtypeStruct((H, NB, S), _F32),
    )(pooled, gq_w, gk_w, expander)


_QKV_BM = 512
_PB = _QKV_BM // BLK


def _rope_piece(x, c, sn):
    x1 = x[:, :HD // 2]
    x2 = x[:, HD // 2:]
    rot = jnp.concatenate([-x2, x1], axis=-1)
    return x * c + rot * sn


def _qkv_body(x_ref, qw_ref, kw_ref, vw_ref, b_ref, cos_ref, sin_ref,
              g_ref, o_ref, p_ref):
    xr = x_ref[...]
    var = jnp.mean(xr * xr, axis=-1, keepdims=True)
    x = ((xr * jax.lax.rsqrt(var + EPS)) * g_ref[...]).astype(_BF16)
    qw = qw_ref[...].astype(_BF16)
    kw = kw_ref[...].astype(_BF16)
    vw = vw_ref[...].astype(_BF16)
    oq = jnp.dot(x, qw, preferred_element_type=_F32)
    ok = jnp.dot(x, kw, preferred_element_type=_F32)
    ov = jnp.dot(x, vw, preferred_element_type=_F32)
    out = jnp.concatenate([oq, ok, ov], axis=-1) + b_ref[...]
    p_ref[...] = out.reshape(_PB, BLK, (H + 2 * KVH) * HD).sum(axis=1)
    c = cos_ref[...]
    sn = sin_ref[...]
    for hh in range(H + KVH):
        piece = out[:, hh * HD:(hh + 1) * HD]
        o_ref[:, hh * HD:(hh + 1) * HD] = _rope_piece(piece, c, sn).astype(_BF16)
    o_ref[:, (H + KVH) * HD:] = out[:, (H + KVH) * HD:].astype(_BF16)


def _qkv(x, qw, kw, vw, b, cos, sin, g):
    n_all = (H + 2 * KVH) * HD
    return pl.pallas_call(
        _qkv_body,
        grid=(S // _QKV_BM,),
        in_specs=[
            pl.BlockSpec((_QKV_BM, D), lambda mm: (mm, 0)),
            pl.BlockSpec((D, H * HD), lambda mm: (0, 0)),
            pl.BlockSpec((D, KVH * HD), lambda mm: (0, 0)),
            pl.BlockSpec((D, KVH * HD), lambda mm: (0, 0)),
            pl.BlockSpec((1, n_all), lambda mm: (0, 0)),
            pl.BlockSpec((_QKV_BM, HD), lambda mm: (mm, 0)),
            pl.BlockSpec((_QKV_BM, HD), lambda mm: (mm, 0)),
            pl.BlockSpec((1, D), lambda mm: (0, 0)),
        ],
        out_specs=[
            pl.BlockSpec((_QKV_BM, n_all), lambda mm: (mm, 0)),
            pl.BlockSpec((_PB, n_all), lambda mm: (mm, 0)),
        ],
        out_shape=[
            jax.ShapeDtypeStruct((S, n_all), _BF16),
            jax.ShapeDtypeStruct((NB, n_all), _F32),
        ],
    )(x, qw, kw, vw, b.reshape(1, n_all), cos, sin, g.reshape(1, D))


def _oproj_body(x_ref, w_ref, r_ref, g_ref, h_ref, xn_ref):
    w = w_ref[...].astype(_BF16)
    acc = r_ref[...] + jnp.dot(x_ref[...], w, preferred_element_type=_F32)
    h_ref[...] = acc
    var = jnp.mean(acc * acc, axis=-1, keepdims=True)
    xn_ref[...] = ((acc * jax.lax.rsqrt(var + EPS)) * g_ref[...]).astype(_BF16)


def _oproj(x, w, r, g, bm=512):
    return pl.pallas_call(
        _oproj_body,
        grid=(S // bm,),
        in_specs=[
            pl.BlockSpec((bm, H * HD), lambda mm: (mm, 0)),
            pl.BlockSpec((H * HD, D), lambda mm: (0, 0)),
            pl.BlockSpec((bm, D), lambda mm: (mm, 0)),
            pl.BlockSpec((1, D), lambda mm: (0, 0)),
        ],
        out_specs=[
            pl.BlockSpec((bm, D), lambda mm: (mm, 0)),
            pl.BlockSpec((bm, D), lambda mm: (mm, 0)),
        ],
        out_shape=[
            jax.ShapeDtypeStruct((S, D), _F32),
            jax.ShapeDtypeStruct((S, D), _BF16),
        ],
    )(x, w, r, g.reshape(1, D))


def _matmul_res_body(x_ref, w_ref, r_ref, o_ref):
    w = w_ref[...].astype(_BF16)
    o_ref[...] = r_ref[...] + jnp.dot(
        x_ref[...], w, preferred_element_type=_F32)


def _matmul_res(x, w, r, bm, bn):
    m, k = x.shape
    n = w.shape[1]
    grid = (pl.cdiv(n, bn), pl.cdiv(m, bm))
    return pl.pallas_call(
        _matmul_res_body,
        grid=grid,
        in_specs=[
            pl.BlockSpec((bm, k), lambda nn, mm: (mm, 0)),
            pl.BlockSpec((k, bn), lambda nn, mm: (0, nn)),
            pl.BlockSpec((bm, bn), lambda nn, mm: (mm, nn)),
        ],
        out_specs=pl.BlockSpec((bm, bn), lambda nn, mm: (mm, nn)),
        out_shape=jax.ShapeDtypeStruct((m, n), _F32),
    )(x, w, r)


def _mlp1_body(x_ref, gw_ref, uw_ref, o_ref):
    x = x_ref[...]
    a = jnp.dot(x, gw_ref[...].astype(_BF16), preferred_element_type=_F32)
    u = jnp.dot(x, uw_ref[...].astype(_BF16), preferred_element_type=_F32)
    o_ref[...] = ((a * jax.nn.sigmoid(a)) * u).astype(_BF16)


def _mlp1(x, gw, uw, bm, bn):
    m, k = x.shape
    n = gw.shape[1]
    grid = (pl.cdiv(n, bn), pl.cdiv(m, bm))
    return pl.pallas_call(
        _mlp1_body,
        grid=grid,
        in_specs=[
            pl.BlockSpec((bm, k), lambda nn, mm: (mm, 0)),
            pl.BlockSpec((k, bn), lambda nn, mm: (0, nn)),
            pl.BlockSpec((k, bn), lambda nn, mm: (0, nn)),
        ],
        out_specs=pl.BlockSpec((bm, bn), lambda nn, mm: (mm, nn)),
        out_shape=jax.ShapeDtypeStruct((m, n), _BF16),
    )(x, gw, uw)


BQ = 256
BQB = BQ // BLK
BKV = 512
MQ = S // BQ
_SCALE = 1.0 / math.sqrt(HD)
_NEG = -1e9


def _flash_body(q_ref, k_ref, v_ref, b_ref, o_ref):
    mi = pl.program_id(1)
    jlast = mi // 2

    for h in range(GQ):
        q = q_ref[:, h * HD:(h + 1) * HD]

        def chunk(jj, carry, causal):
            m_prev, l_prev, acc = carry
            kc = k_ref[pl.ds(jj * BKV, BKV), :]
            s = jax.lax.dot_general(
                q, kc, (((1,), (1,)), ((), ())),
                preferred_element_type=_F32) * _SCALE
            bc = b_ref[0, h, 0, :, pl.ds(jj * BKV, BKV)]
            s = (s.reshape(BQB, BLK, BKV) + bc[:, None, :]).reshape(BQ, BKV)
            if causal:
                rows = mi * BQ + jax.lax.broadcasted_iota(
                    jnp.int32, (BQ, BKV), 0)
                cols = jj * BKV + jax.lax.broadcasted_iota(
                    jnp.int32, (BQ, BKV), 1)
                s = jnp.where(cols <= rows, s, _NEG)
            m_new = jnp.maximum(m_prev, jnp.max(s, axis=-1, keepdims=True))
            p = jnp.exp(s - m_new)
            alpha = jnp.exp(m_prev - m_new)
            l_new = l_prev * alpha + jnp.sum(p, axis=-1, keepdims=True)
            vc = v_ref[pl.ds(jj * BKV, BKV), :]
            acc_new = acc * alpha + jnp.dot(
                p.astype(_BF16), vc, preferred_element_type=_F32)
            return m_new, l_new, acc_new

        init = (jnp.full((BQ, 1), -1e30, _F32),
                jnp.zeros((BQ, 1), _F32),
                jnp.zeros((BQ, HD), _F32))
        carry = jax.lax.fori_loop(
            0, jlast, lambda jj, c: chunk(jj, c, causal=False), init)
        _, l_fin, acc_fin = chunk(jlast, carry, causal=True)
        o_ref[:, h * HD:(h + 1) * HD] = (acc_fin / l_fin).astype(_BF16)


def _flash(qkv, bias):
    grid = (KVH, MQ)
    return pl.pallas_call(
        _flash_body,
        grid=grid,
        in_specs=[
            pl.BlockSpec((BQ, GQ * HD), lambda g, m: (m, g)),
            pl.BlockSpec((S, HD), lambda g, m: (0, H + g)),
            pl.BlockSpec((S, HD), lambda g, m: (0, H + KVH + g)),
            pl.BlockSpec((1, GQ, 1, BQB, S), lambda g, m: (g, 0, m, 0, 0)),
        ],
        out_specs=pl.BlockSpec((BQ, GQ * HD), lambda g, m: (m, g)),
        out_shape=jax.ShapeDtypeStruct((S, H * HD), _BF16),
    )(qkv, qkv, qkv, bias)


def _rope_tables(position_ids):
    inv_freq = 1.0 / (THETA ** (jnp.arange(0, HD, 2, dtype=_F32) / HD))
    freqs = position_ids[0].astype(_F32)[:, None] * inv_freq[None, :]
    emb = jnp.concatenate([freqs, freqs], axis=-1)
    return jnp.cos(emb), jnp.sin(emb)


def kernel(hidden_states, position_ids, ln1_w, q_w, q_b, k_w, k_b, v_w, v_b,
           o_w, gq_w, gk_w, ln2_w, gate_w, up_w, down_w):
    hs = hidden_states.reshape(S, D)

    cos, sin = _rope_tables(position_ids)
    bqkv = jnp.concatenate([q_b, k_b, v_b], axis=0)
    qkv, pooled = _qkv(hs, q_w, k_w, v_w, bqkv, cos, sin, ln1_w)

    expander = jnp.equal(jnp.arange(S, dtype=jnp.int32)[None, :] // BLK,
                         jnp.arange(NB, dtype=jnp.int32)[:, None]
                         ).astype(_F32)
    bias_tok = _gate(pooled, gq_w, gk_w, expander)
    bias5 = bias_tok.reshape(KVH, GQ, MQ, BQB, S)

    attn2 = _flash(qkv, bias5)

    hidden, xn2 = _oproj(attn2, o_w, hs, ln2_w)
    mlp_mid = _mlp1(xn2, gate_w, up_w, bm=1024, bn=512)
    out = _matmul_res(mlp_mid, down_w, hidden, bm=512, bn=512)
    return out.reshape(1, S, D)


# speedup vs baseline: 1.0703x; 1.0147x over previous
import functools
import math

import jax
import jax.numpy as jnp
import numpy as np
from jax.experimental import pallas as pl
from jax.experimental.pallas import tpu as pltpu

S, D = 2048, 2048
H, KVH, HD = 16, 4, 128
GQ = H // KVH
BLK = 64
GH = 128
I = 5504
EPS = 1e-6
THRESH = 1e-3
THETA = 10000.0
NB = S // BLK

_F32 = jnp.float32
_BF16 = jnp.bfloat16


def _gate_body(p_ref, gqw_ref, gkw_ref, e_ref, o_ref):
    inv = 1.0 / BLK
    gkw = gkw_ref[...]
    gqw = gqw_ref[...]
    ex = e_ref[...]
    kgs = []
    for g in range(KVH):
        pk = p_ref[:, (H + g) * HD:(H + g + 1) * HD] * inv
        kgs.append(jnp.dot(pk, gkw, preferred_element_type=_F32))
    ri = jax.lax.broadcasted_iota(jnp.int32, (NB, NB), 0)
    ci = jax.lax.broadcasted_iota(jnp.int32, (NB, NB), 1)
    causal = ci <= ri
    diag = ci == ri
    for h in range(H):
        pq = p_ref[:, h * HD:(h + 1) * HD] * inv
        qg = jnp.dot(pq, gqw, preferred_element_type=_F32)
        lg = jax.lax.dot_general(
            qg, kgs[h // GQ], (((1,), (1,)), ((), ())),
            preferred_element_type=_F32) * (1.0 / math.sqrt(GH))
        lg = jnp.where(causal, lg, _NEG)
        mx = jnp.max(lg, axis=-1, keepdims=True)
        pe = jnp.exp(lg - mx)
        score = pe / jnp.sum(pe, axis=-1, keepdims=True)
        mask = ((score >= THRESH) | diag) & causal
        bias = jnp.where(mask, 0.0, _NEG)
        o_ref[h] = jnp.dot(bias, ex, preferred_element_type=_F32)


def _gate(pooled, gq_w, gk_w, expander):
    n_all = (H + 2 * KVH) * HD
    return pl.pallas_call(
        _gate_body,
        grid=(1,),
        in_specs=[
            pl.BlockSpec((NB, n_all), lambda i: (0, 0)),
            pl.BlockSpec((HD, GH), lambda i: (0, 0)),
            pl.BlockSpec((HD, GH), lambda i: (0, 0)),
            pl.BlockSpec((NB, S), lambda i: (0, 0)),
        ],
        out_specs=pl.BlockSpec((H, NB, S), lambda i: (0, 0, 0)),
        out_shape=jax.ShapeDtypeStruct((H, NB, S), _F32),
    )(pooled, gq_w, gk_w, expander)


_QKV_BM = 512
_PB = _QKV_BM // BLK


def _rope_piece(x, c, sn):
    x1 = x[:, :HD // 2]
    x2 = x[:, HD // 2:]
    rot = jnp.concatenate([-x2, x1], axis=-1)
    return x * c + rot * sn


def _qkv_body(x_ref, qw_ref, kw_ref, vw_ref, b_ref, cos_ref, sin_ref,
              g_ref, o_ref, p_ref):
    xr = x_ref[...]
    var = jnp.mean(xr * xr, axis=-1, keepdims=True)
    x = ((xr * jax.lax.rsqrt(var + EPS)) * g_ref[...]).astype(_BF16)
    qw = qw_ref[...].astype(_BF16)
    kw = kw_ref[...].astype(_BF16)
    vw = vw_ref[...].astype(_BF16)
    oq = jnp.dot(x, qw, preferred_element_type=_F32)
    ok = jnp.dot(x, kw, preferred_element_type=_F32)
    ov = jnp.dot(x, vw, preferred_element_type=_F32)
    out = jnp.concatenate([oq, ok, ov], axis=-1) + b_ref[...]
    p_ref[...] = out.reshape(_PB, BLK, (H + 2 * KVH) * HD).sum(axis=1)
    c = cos_ref[...]
    sn = sin_ref[...]
    cs = c * _SCALE
    sns = sn * _SCALE
    for hh in range(H):
        piece = out[:, hh * HD:(hh + 1) * HD]
        o_ref[:, hh * HD:(hh + 1) * HD] = _rope_piece(piece, cs, sns).astype(_BF16)
    for hh in range(H, H + KVH):
        piece = out[:, hh * HD:(hh + 1) * HD]
        o_ref[:, hh * HD:(hh + 1) * HD] = _rope_piece(piece, c, sn).astype(_BF16)
    o_ref[:, (H + KVH) * HD:] = out[:, (H + KVH) * HD:].astype(_BF16)


def _qkv(x, qw, kw, vw, b, cos, sin, g):
    n_all = (H + 2 * KVH) * HD
    return pl.pallas_call(
        _qkv_body,
        grid=(S // _QKV_BM,),
        in_specs=[
            pl.BlockSpec((_QKV_BM, D), lambda mm: (mm, 0)),
            pl.BlockSpec((D, H * HD), lambda mm: (0, 0)),
            pl.BlockSpec((D, KVH * HD), lambda mm: (0, 0)),
            pl.BlockSpec((D, KVH * HD), lambda mm: (0, 0)),
            pl.BlockSpec((1, n_all), lambda mm: (0, 0)),
            pl.BlockSpec((_QKV_BM, HD), lambda mm: (mm, 0)),
            pl.BlockSpec((_QKV_BM, HD), lambda mm: (mm, 0)),
            pl.BlockSpec((1, D), lambda mm: (0, 0)),
        ],
        out_specs=[
            pl.BlockSpec((_QKV_BM, n_all), lambda mm: (mm, 0)),
            pl.BlockSpec((_PB, n_all), lambda mm: (mm, 0)),
        ],
        out_shape=[
            jax.ShapeDtypeStruct((S, n_all), _BF16),
            jax.ShapeDtypeStruct((NB, n_all), _F32),
        ],
    )(x, qw, kw, vw, b.reshape(1, n_all), cos, sin, g.reshape(1, D))


def _oproj_body(x_ref, w_ref, r_ref, g_ref, h_ref, xn_ref):
    w = w_ref[...].astype(_BF16)
    acc = r_ref[...] + jnp.dot(x_ref[...], w, preferred_element_type=_F32)
    h_ref[...] = acc
    var = jnp.mean(acc * acc, axis=-1, keepdims=True)
    xn_ref[...] = ((acc * jax.lax.rsqrt(var + EPS)) * g_ref[...]).astype(_BF16)


def _oproj(x, w, r, g, bm=512):
    return pl.pallas_call(
        _oproj_body,
        grid=(S // bm,),
        in_specs=[
            pl.BlockSpec((bm, H * HD), lambda mm: (mm, 0)),
            pl.BlockSpec((H * HD, D), lambda mm: (0, 0)),
            pl.BlockSpec((bm, D), lambda mm: (mm, 0)),
            pl.BlockSpec((1, D), lambda mm: (0, 0)),
        ],
        out_specs=[
            pl.BlockSpec((bm, D), lambda mm: (mm, 0)),
            pl.BlockSpec((bm, D), lambda mm: (mm, 0)),
        ],
        out_shape=[
            jax.ShapeDtypeStruct((S, D), _F32),
            jax.ShapeDtypeStruct((S, D), _BF16),
        ],
    )(x, w, r, g.reshape(1, D))


def _matmul_res_body(x_ref, w_ref, r_ref, o_ref):
    w = w_ref[...].astype(_BF16)
    o_ref[...] = r_ref[...] + jnp.dot(
        x_ref[...], w, preferred_element_type=_F32)


def _matmul_res(x, w, r, bm, bn):
    m, k = x.shape
    n = w.shape[1]
    grid = (pl.cdiv(n, bn), pl.cdiv(m, bm))
    return pl.pallas_call(
        _matmul_res_body,
        grid=grid,
        in_specs=[
            pl.BlockSpec((bm, k), lambda nn, mm: (mm, 0)),
            pl.BlockSpec((k, bn), lambda nn, mm: (0, nn)),
            pl.BlockSpec((bm, bn), lambda nn, mm: (mm, nn)),
        ],
        out_specs=pl.BlockSpec((bm, bn), lambda nn, mm: (mm, nn)),
        out_shape=jax.ShapeDtypeStruct((m, n), _F32),
    )(x, w, r)


def _mlp1_body(x_ref, gw_ref, uw_ref, o_ref):
    x = x_ref[...]
    a = jnp.dot(x, gw_ref[...].astype(_BF16), preferred_element_type=_F32)
    u = jnp.dot(x, uw_ref[...].astype(_BF16), preferred_element_type=_F32)
    o_ref[...] = ((a * jax.nn.sigmoid(a)) * u).astype(_BF16)


def _mlp1(x, gw, uw, bm, bn):
    m, k = x.shape
    n = gw.shape[1]
    grid = (pl.cdiv(n, bn), pl.cdiv(m, bm))
    return pl.pallas_call(
        _mlp1_body,
        grid=grid,
        in_specs=[
            pl.BlockSpec((bm, k), lambda nn, mm: (mm, 0)),
            pl.BlockSpec((k, bn), lambda nn, mm: (0, nn)),
            pl.BlockSpec((k, bn), lambda nn, mm: (0, nn)),
        ],
        out_specs=pl.BlockSpec((bm, bn), lambda nn, mm: (mm, nn)),
        out_shape=jax.ShapeDtypeStruct((m, n), _BF16),
    )(x, gw, uw)


BQ = 256
BQB = BQ // BLK
BKV = 512
MQ = S // BQ
_SCALE = 1.0 / math.sqrt(HD)
_NEG = -1e9


def _flash_body(q_ref, k_ref, v_ref, b_ref, o_ref):
    mi = pl.program_id(1)
    jlast = mi // 2

    for h in range(GQ):
        q = q_ref[:, h * HD:(h + 1) * HD]

        def chunk(jj, carry, causal):
            m_prev, l_prev, acc = carry
            kc = k_ref[pl.ds(jj * BKV, BKV), :]
            s = jax.lax.dot_general(
                q, kc, (((1,), (1,)), ((), ())),
                preferred_element_type=_F32)
            bc = b_ref[0, h, 0, :, pl.ds(jj * BKV, BKV)]
            s = (s.reshape(BQB, BLK, BKV) + bc[:, None, :]).reshape(BQ, BKV)
            if causal:
                rows = mi * BQ + jax.lax.broadcasted_iota(
                    jnp.int32, (BQ, BKV), 0)
                cols = jj * BKV + jax.lax.broadcasted_iota(
                    jnp.int32, (BQ, BKV), 1)
                s = jnp.where(cols <= rows, s, _NEG)
            m_new = jnp.maximum(m_prev, jnp.max(s, axis=-1, keepdims=True))
            p = jnp.exp(s - m_new)
            alpha = jnp.exp(m_prev - m_new)
            l_new = l_prev * alpha + jnp.sum(p, axis=-1, keepdims=True)
            vc = v_ref[pl.ds(jj * BKV, BKV), :]
            acc_new = acc * alpha + jnp.dot(
                p.astype(_BF16), vc, preferred_element_type=_F32)
            return m_new, l_new, acc_new

        init = (jnp.full((BQ, 1), -1e30, _F32),
                jnp.zeros((BQ, 1), _F32),
                jnp.zeros((BQ, HD), _F32))
        carry = jax.lax.fori_loop(
            0, jlast, lambda jj, c: chunk(jj, c, causal=False), init)
        _, l_fin, acc_fin = chunk(jlast, carry, causal=True)
        o_ref[:, h * HD:(h + 1) * HD] = (acc_fin / l_fin).astype(_BF16)


def _flash(qkv, bias):
    grid = (KVH, MQ)
    return pl.pallas_call(
        _flash_body,
        grid=grid,
        in_specs=[
            pl.BlockSpec((BQ, GQ * HD), lambda g, m: (m, g)),
            pl.BlockSpec((S, HD), lambda g, m: (0, H + g)),
            pl.BlockSpec((S, HD), lambda g, m: (0, H + KVH + g)),
            pl.BlockSpec((1, GQ, 1, BQB, S), lambda g, m: (g, 0, m, 0, 0)),
        ],
        out_specs=pl.BlockSpec((BQ, GQ * HD), lambda g, m: (m, g)),
        out_shape=jax.ShapeDtypeStruct((S, H * HD), _BF16),
    )(qkv, qkv, qkv, bias)


def _rope_tables(position_ids):
    inv_freq = 1.0 / (THETA ** (jnp.arange(0, HD, 2, dtype=_F32) / HD))
    freqs = position_ids[0].astype(_F32)[:, None] * inv_freq[None, :]
    emb = jnp.concatenate([freqs, freqs], axis=-1)
    return jnp.cos(emb), jnp.sin(emb)


def kernel(hidden_states, position_ids, ln1_w, q_w, q_b, k_w, k_b, v_w, v_b,
           o_w, gq_w, gk_w, ln2_w, gate_w, up_w, down_w):
    hs = hidden_states.reshape(S, D)

    cos, sin = _rope_tables(position_ids)
    bqkv = jnp.concatenate([q_b, k_b, v_b], axis=0)
    qkv, pooled = _qkv(hs, q_w, k_w, v_w, bqkv, cos, sin, ln1_w)

    expander = jnp.equal(jnp.arange(S, dtype=jnp.int32)[None, :] // BLK,
                         jnp.arange(NB, dtype=jnp.int32)[:, None]
                         ).astype(_F32)
    bias_tok = _gate(pooled, gq_w, gk_w, expander)
    bias5 = bias_tok.reshape(KVH, GQ, MQ, BQB, S)

    attn2 = _flash(qkv, bias5)

    hidden, xn2 = _oproj(attn2, o_w, hs, ln2_w)
    mlp_mid = _mlp1(xn2, gate_w, up_w, bm=2048, bn=512)
    out = _matmul_res(mlp_mid, down_w, hidden, bm=2048, bn=256)
    return out.reshape(1, S, D)


# speedup vs baseline: 1.0768x; 1.0061x over previous
import math

import jax
import jax.numpy as jnp
from jax.experimental import pallas as pl

S, D = 2048, 2048
H, KVH, HD = 16, 4, 128
GQ = H // KVH
BLK = 64
GH = 128
I = 5504
EPS = 1e-6
THRESH = 1e-3
THETA = 10000.0
NB = S // BLK

_F32 = jnp.float32
_BF16 = jnp.bfloat16


def _gate_body(p_ref, gqw_ref, gkw_ref, e_ref, o_ref):
    inv = 1.0 / BLK
    gkw = gkw_ref[...]
    gqw = gqw_ref[...]
    ex = e_ref[...]
    kgs = []
    for g in range(KVH):
        pk = p_ref[:, (H + g) * HD:(H + g + 1) * HD] * inv
        kgs.append(jnp.dot(pk, gkw, preferred_element_type=_F32))
    ri = jax.lax.broadcasted_iota(jnp.int32, (NB, NB), 0)
    ci = jax.lax.broadcasted_iota(jnp.int32, (NB, NB), 1)
    causal = ci <= ri
    diag = ci == ri
    for h in range(H):
        pq = p_ref[:, h * HD:(h + 1) * HD] * inv
        qg = jnp.dot(pq, gqw, preferred_element_type=_F32)
        lg = jax.lax.dot_general(
            qg, kgs[h // GQ], (((1,), (1,)), ((), ())),
            preferred_element_type=_F32) * (1.0 / math.sqrt(GH))
        lg = jnp.where(causal, lg, _NEG)
        mx = jnp.max(lg, axis=-1, keepdims=True)
        pe = jnp.exp(lg - mx)
        score = pe / jnp.sum(pe, axis=-1, keepdims=True)
        mask = ((score >= THRESH) | diag) & causal
        bias = jnp.where(mask, 0.0, _NEG)
        o_ref[h] = jnp.dot(bias, ex, preferred_element_type=_F32)


def _gate(pooled, gq_w, gk_w, expander):
    n_all = (H + 2 * KVH) * HD
    return pl.pallas_call(
        _gate_body,
        grid=(1,),
        in_specs=[
            pl.BlockSpec((NB, n_all), lambda i: (0, 0)),
            pl.BlockSpec((HD, GH), lambda i: (0, 0)),
            pl.BlockSpec((HD, GH), lambda i: (0, 0)),
            pl.BlockSpec((NB, S), lambda i: (0, 0)),
        ],
        out_specs=pl.BlockSpec((H, NB, S), lambda i: (0, 0, 0)),
        out_shape=jax.ShapeDtypeStruct((H, NB, S), _F32),
    )(pooled, gq_w, gk_w, expander)


_QKV_BM = 512
_PB = _QKV_BM // BLK


def _rope_piece(x, c, sn):
    x1 = x[:, :HD // 2]
    x2 = x[:, HD // 2:]
    rot = jnp.concatenate([-x2, x1], axis=-1)
    return x * c + rot * sn


def _qkv_body(x_ref, qw_ref, kw_ref, vw_ref, b_ref, cos_ref, sin_ref,
              g_ref, o_ref, p_ref):
    xr = x_ref[...]
    var = jnp.mean(xr * xr, axis=-1, keepdims=True)
    x = ((xr * jax.lax.rsqrt(var + EPS)) * g_ref[...]).astype(_BF16)
    qw = qw_ref[...].astype(_BF16)
    kw = kw_ref[...].astype(_BF16)
    vw = vw_ref[...].astype(_BF16)
    oq = jnp.dot(x, qw, preferred_element_type=_F32)
    ok = jnp.dot(x, kw, preferred_element_type=_F32)
    ov = jnp.dot(x, vw, preferred_element_type=_F32)
    out = jnp.concatenate([oq, ok, ov], axis=-1) + b_ref[...]
    p_ref[...] = out.reshape(_PB, BLK, (H + 2 * KVH) * HD).sum(axis=1)
    c = cos_ref[...]
    sn = sin_ref[...]
    cs = c * _SCALE
    sns = sn * _SCALE
    for hh in range(H):
        piece = out[:, hh * HD:(hh + 1) * HD]
        o_ref[:, hh * HD:(hh + 1) * HD] = _rope_piece(piece, cs, sns).astype(_BF16)
    for hh in range(H, H + KVH):
        piece = out[:, hh * HD:(hh + 1) * HD]
        o_ref[:, hh * HD:(hh + 1) * HD] = _rope_piece(piece, c, sn).astype(_BF16)
    o_ref[:, (H + KVH) * HD:] = out[:, (H + KVH) * HD:].astype(_BF16)


def _qkv(x, qw, kw, vw, b, cos, sin, g):
    n_all = (H + 2 * KVH) * HD
    return pl.pallas_call(
        _qkv_body,
        grid=(S // _QKV_BM,),
        in_specs=[
            pl.BlockSpec((_QKV_BM, D), lambda mm: (mm, 0)),
            pl.BlockSpec((D, H * HD), lambda mm: (0, 0)),
            pl.BlockSpec((D, KVH * HD), lambda mm: (0, 0)),
            pl.BlockSpec((D, KVH * HD), lambda mm: (0, 0)),
            pl.BlockSpec((1, n_all), lambda mm: (0, 0)),
            pl.BlockSpec((_QKV_BM, HD), lambda mm: (mm, 0)),
            pl.BlockSpec((_QKV_BM, HD), lambda mm: (mm, 0)),
            pl.BlockSpec((1, D), lambda mm: (0, 0)),
        ],
        out_specs=[
            pl.BlockSpec((_QKV_BM, n_all), lambda mm: (mm, 0)),
            pl.BlockSpec((_PB, n_all), lambda mm: (mm, 0)),
        ],
        out_shape=[
            jax.ShapeDtypeStruct((S, n_all), _BF16),
            jax.ShapeDtypeStruct((NB, n_all), _F32),
        ],
    )(x, qw, kw, vw, b.reshape(1, n_all), cos, sin, g.reshape(1, D))


def _oproj_body(x_ref, w_ref, r_ref, g_ref, h_ref, xn_ref):
    w = w_ref[...].astype(_BF16)
    acc = r_ref[...] + jnp.dot(x_ref[...], w, preferred_element_type=_F32)
    h_ref[...] = acc
    var = jnp.mean(acc * acc, axis=-1, keepdims=True)
    xn_ref[...] = ((acc * jax.lax.rsqrt(var + EPS)) * g_ref[...]).astype(_BF16)


def _oproj(x, w, r, g, bm=512):
    return pl.pallas_call(
        _oproj_body,
        grid=(S // bm,),
        in_specs=[
            pl.BlockSpec((bm, H * HD), lambda mm: (mm, 0)),
            pl.BlockSpec((H * HD, D), lambda mm: (0, 0)),
            pl.BlockSpec((bm, D), lambda mm: (mm, 0)),
            pl.BlockSpec((1, D), lambda mm: (0, 0)),
        ],
        out_specs=[
            pl.BlockSpec((bm, D), lambda mm: (mm, 0)),
            pl.BlockSpec((bm, D), lambda mm: (mm, 0)),
        ],
        out_shape=[
            jax.ShapeDtypeStruct((S, D), _F32),
            jax.ShapeDtypeStruct((S, D), _BF16),
        ],
    )(x, w, r, g.reshape(1, D))


def _matmul_res_body(x_ref, w_ref, r_ref, o_ref):
    w = w_ref[...].astype(_BF16)
    o_ref[...] = r_ref[...] + jnp.dot(
        x_ref[...], w, preferred_element_type=_F32)


def _matmul_res(x, w, r, bm, bn):
    m, k = x.shape
    n = w.shape[1]
    grid = (pl.cdiv(n, bn), pl.cdiv(m, bm))
    return pl.pallas_call(
        _matmul_res_body,
        grid=grid,
        in_specs=[
            pl.BlockSpec((bm, k), lambda nn, mm: (mm, 0)),
            pl.BlockSpec((k, bn), lambda nn, mm: (0, nn)),
            pl.BlockSpec((bm, bn), lambda nn, mm: (mm, nn)),
        ],
        out_specs=pl.BlockSpec((bm, bn), lambda nn, mm: (mm, nn)),
        out_shape=jax.ShapeDtypeStruct((m, n), _F32),
    )(x, w, r)


def _mlp1_body(x_ref, gw_ref, uw_ref, o_ref):
    x = x_ref[...]
    a = jnp.dot(x, gw_ref[...].astype(_BF16), preferred_element_type=_F32)
    u = jnp.dot(x, uw_ref[...].astype(_BF16), preferred_element_type=_F32)
    o_ref[...] = ((a * jax.nn.sigmoid(a)) * u).astype(_BF16)


def _mlp1(x, gw, uw, bm, bn):
    m, k = x.shape
    n = gw.shape[1]
    grid = (pl.cdiv(n, bn), pl.cdiv(m, bm))
    return pl.pallas_call(
        _mlp1_body,
        grid=grid,
        in_specs=[
            pl.BlockSpec((bm, k), lambda nn, mm: (mm, 0)),
            pl.BlockSpec((k, bn), lambda nn, mm: (0, nn)),
            pl.BlockSpec((k, bn), lambda nn, mm: (0, nn)),
        ],
        out_specs=pl.BlockSpec((bm, bn), lambda nn, mm: (mm, nn)),
        out_shape=jax.ShapeDtypeStruct((m, n), _BF16),
    )(x, gw, uw)


BQ = 256
BQB = BQ // BLK
BKV = 512
MQ = S // BQ
_SCALE = 1.0 / math.sqrt(HD)
_NEG = -1e9


def _flash_body(q_ref, k_ref, v_ref, b_ref, o_ref):
    mi = pl.program_id(1)
    jlast = mi // 2

    for h in range(GQ):
        q = q_ref[:, h * HD:(h + 1) * HD]

        def chunk(jj, carry, causal):
            m_prev, l_prev, acc = carry
            kc = k_ref[pl.ds(jj * BKV, BKV), :]
            s = jax.lax.dot_general(
                q, kc, (((1,), (1,)), ((), ())),
                preferred_element_type=_F32)
            bc = b_ref[0, h, 0, :, pl.ds(jj * BKV, BKV)]
            s = (s.reshape(BQB, BLK, BKV) + bc[:, None, :]).reshape(BQ, BKV)
            if causal:
                rows = mi * BQ + jax.lax.broadcasted_iota(
                    jnp.int32, (BQ, BKV), 0)
                cols = jj * BKV + jax.lax.broadcasted_iota(
                    jnp.int32, (BQ, BKV), 1)
                s = jnp.where(cols <= rows, s, _NEG)
            m_new = jnp.maximum(m_prev, jnp.max(s, axis=-1, keepdims=True))
            p = jnp.exp(s - m_new)
            alpha = jnp.exp(m_prev - m_new)
            l_new = l_prev * alpha + jnp.sum(p, axis=-1, keepdims=True)
            vc = v_ref[pl.ds(jj * BKV, BKV), :]
            acc_new = acc * alpha + jnp.dot(
                p.astype(_BF16), vc, preferred_element_type=_F32)
            return m_new, l_new, acc_new

        init = (jnp.full((BQ, 1), -1e30, _F32),
                jnp.zeros((BQ, 1), _F32),
                jnp.zeros((BQ, HD), _F32))
        carry = jax.lax.fori_loop(
            0, jlast, lambda jj, c: chunk(jj, c, causal=False), init)
        _, l_fin, acc_fin = chunk(jlast, carry, causal=True)
        o_ref[:, h * HD:(h + 1) * HD] = (acc_fin / l_fin).astype(_BF16)


def _flash(qkv, bias):
    grid = (KVH, MQ)
    return pl.pallas_call(
        _flash_body,
        grid=grid,
        in_specs=[
            pl.BlockSpec((BQ, GQ * HD), lambda g, m: (m, g)),
            pl.BlockSpec((S, HD), lambda g, m: (0, H + g)),
            pl.BlockSpec((S, HD), lambda g, m: (0, H + KVH + g)),
            pl.BlockSpec((1, GQ, 1, BQB, S), lambda g, m: (g, 0, m, 0, 0)),
        ],
        out_specs=pl.BlockSpec((BQ, GQ * HD), lambda g, m: (m, g)),
        out_shape=jax.ShapeDtypeStruct((S, H * HD), _BF16),
    )(qkv, qkv, qkv, bias)


def _rope_tables(position_ids):
    inv_freq = 1.0 / (THETA ** (jnp.arange(0, HD, 2, dtype=_F32) / HD))
    freqs = position_ids[0].astype(_F32)[:, None] * inv_freq[None, :]
    emb = jnp.concatenate([freqs, freqs], axis=-1)
    return jnp.cos(emb), jnp.sin(emb)


def kernel(hidden_states, position_ids, ln1_w, q_w, q_b, k_w, k_b, v_w, v_b,
           o_w, gq_w, gk_w, ln2_w, gate_w, up_w, down_w):
    hs = hidden_states.reshape(S, D)

    cos, sin = _rope_tables(position_ids)
    bqkv = jnp.concatenate([q_b, k_b, v_b], axis=0)
    qkv, pooled = _qkv(hs, q_w, k_w, v_w, bqkv, cos, sin, ln1_w)

    expander = jnp.equal(jnp.arange(S, dtype=jnp.int32)[None, :] // BLK,
                         jnp.arange(NB, dtype=jnp.int32)[:, None]
                         ).astype(_F32)
    bias_tok = _gate(pooled, gq_w, gk_w, expander)
    bias5 = bias_tok.reshape(KVH, GQ, MQ, BQB, S)

    attn2 = _flash(qkv, bias5)

    hidden, xn2 = _oproj(attn2, o_w, hs, ln2_w)
    mlp_mid = _mlp1(xn2, gate_w, up_w, bm=2048, bn=512)
    out = _matmul_res(mlp_mid, down_w, hidden, bm=2048, bn=256)
    return out.reshape(1, S, D)
